# baseline (speedup 1.0000x reference)
import functools

import jax
import jax.numpy as jnp
from jax import lax
from jax.experimental import pallas as pl
from jax.experimental.pallas import tpu as pltpu
from jax.experimental.pallas import tpu_sc as plsc

DIM = 64
N_TOK = 819200
NW = 32
BW = N_TOK // NW
CHUNK = 128
NCHUNK = BW // CHUNK
BLK = 40000


def _sc_hist(x2d, zeros_hbm, nbins):
    mesh = plsc.VectorSubcoreMesh(core_axis_name="c", subcore_axis_name="s")

    @functools.partial(
        pl.kernel,
        mesh=mesh,
        out_type=jax.ShapeDtypeStruct((2, nbins), jnp.float32),
        scratch_types=[
            pltpu.VMEM((NCHUNK, CHUNK), jnp.int32),
            pltpu.VMEM((CHUNK,), jnp.float32),
            pltpu.VMEM_SHARED((nbins,), jnp.float32),
            pltpu.SemaphoreType.DMA,
        ],
        compiler_params=pltpu.CompilerParams(use_tc_tiling_on_sc=False),
    )
    def body(x_hbm, z_hbm, out_hbm, idx_v, ones_v, counts_sh, sem):
        cid = lax.axis_index("c")
        sid = lax.axis_index("s")
        wid = sid * 2 + cid
        pltpu.sync_copy(x_hbm.at[pl.ds(wid * NCHUNK, NCHUNK)], idx_v)
        for c in range(CHUNK // 16):
            ones_v[pl.ds(c * 16, 16)] = jnp.ones((16,), jnp.float32)

        @pl.when(sid == 0)
        def _():
            pltpu.sync_copy(z_hbm, counts_sh)

        plsc.subcore_barrier()

        def g_body(g, carry):
            pltpu.async_copy(ones_v, counts_sh.at[idx_v.at[g]], sem, add=True)
            return carry

        lax.fori_loop(0, NCHUNK, g_body, 0)

        def d_body(g, carry):
            pltpu.make_async_copy(
                ones_v, counts_sh.at[idx_v.at[0]], sem).wait()
            return carry

        lax.fori_loop(0, NCHUNK, d_body, 0)
        plsc.subcore_barrier()

        @pl.when(sid == 0)
        def _():
            pltpu.sync_copy(counts_sh, out_hbm.at[cid])

    return body(x2d, zeros_hbm)


LNB = 65536


def _tc_dot(h2, vt, w, b2d):
    nbins = vt.shape[1]
    nblk = pl.cdiv(nbins, LNB)

    nsl = LNB // 128

    def body(h_ref, vt_ref, w_ref, b_ref, o_ref, acc_ref):
        i = pl.program_id(0)

        @pl.when(i == 0)
        def _():
            acc_ref[...] = jnp.zeros_like(acc_ref)

        def accum(masked):
            ps = [jnp.zeros((DIM, 128), jnp.float32) for _ in range(2)]
            for k in range(nsl):
                sl = slice(k * 128, (k + 1) * 128)
                term = vt_ref[:, sl] * (h_ref[0:1, sl] + h_ref[1:2, sl])
                if masked:
                    lane = (lax.broadcasted_iota(jnp.int32, (DIM, 128), 1)
                            + (i * LNB + k * 128))
                    term = jnp.where(lane < nbins, term, 0.0)
                ps[k % 2] = ps[k % 2] + term
            acc_ref[...] += ps[0] + ps[1]

        @pl.when(i != nblk - 1)
        def _():
            accum(False)

        @pl.when(i == nblk - 1)
        def _():
            accum(True)
            vsum = jnp.sum(acc_ref[...], axis=1, keepdims=True)
            hrelu = jnp.maximum(vsum, 0.0)
            logits = lax.dot_general(
                w_ref[...], hrelu, (((1,), (0,)), ((), ())),
                precision=lax.Precision.HIGHEST,
                preferred_element_type=jnp.float32,
            ) + b_ref[...]
            m = jnp.max(logits, axis=0, keepdims=True)
            lse = m + jnp.log(
                jnp.sum(jnp.exp(logits - m), axis=0, keepdims=True))
            o_ref[...] = logits - lse

    return pl.pallas_call(
        body,
        grid=(nblk,),
        in_specs=[
            pl.BlockSpec((2, LNB), lambda i: (0, i)),
            pl.BlockSpec((DIM, LNB), lambda i: (0, i)),
            pl.BlockSpec((2, DIM), lambda i: (0, 0)),
            pl.BlockSpec((2, 1), lambda i: (0, 0)),
        ],
        out_specs=pl.BlockSpec((2, 1), lambda i: (0, 0)),
        out_shape=jax.ShapeDtypeStruct((2, 1), jnp.float32),
        scratch_shapes=[pltpu.VMEM((DIM, 128), jnp.float32)],
        compiler_params=pltpu.CompilerParams(
            vmem_limit_bytes=56 * 1024 * 1024),
    )(h2, vt, w, b2d)


def kernel(X, V, W, b):
    nbins = V.shape[0]
    x2d = X.reshape(NW * NCHUNK, CHUNK)
    zeros = jnp.zeros((nbins,), jnp.float32)
    h2 = _sc_hist(x2d, zeros, nbins)
    out = _tc_dot(h2, V.T, W, b.reshape(2, 1))
    return out.reshape(2)


# speedup vs baseline: 1.0164x; 1.0164x over previous
import functools

import jax
import jax.numpy as jnp
from jax import lax
from jax.experimental import pallas as pl
from jax.experimental.pallas import tpu as pltpu
from jax.experimental.pallas import tpu_sc as plsc

DIM = 64
N_TOK = 819200
NW = 32
BW = N_TOK // NW
CHUNK = 128
NCHUNK = BW // CHUNK
BLK = 40000


def _sc_hist(x2d, zeros_hbm, nbins):
    mesh = plsc.VectorSubcoreMesh(core_axis_name="c", subcore_axis_name="s")

    @functools.partial(
        pl.kernel,
        mesh=mesh,
        out_type=jax.ShapeDtypeStruct((2, nbins), jnp.float32),
        scratch_types=[
            pltpu.VMEM((NCHUNK, CHUNK), jnp.int32),
            pltpu.VMEM((CHUNK,), jnp.float32),
            pltpu.VMEM_SHARED((nbins,), jnp.float32),
            pltpu.SemaphoreType.DMA,
        ],
        compiler_params=pltpu.CompilerParams(use_tc_tiling_on_sc=False),
    )
    def body(x_hbm, z_hbm, out_hbm, idx_v, ones_v, counts_sh, sem):
        cid = lax.axis_index("c")
        sid = lax.axis_index("s")
        wid = sid * 2 + cid
        pltpu.sync_copy(x_hbm.at[pl.ds(wid * NCHUNK, NCHUNK)], idx_v)
        for c in range(CHUNK // 16):
            ones_v[pl.ds(c * 16, 16)] = jnp.ones((16,), jnp.float32)

        @pl.when(sid == 0)
        def _():
            pltpu.sync_copy(z_hbm, counts_sh)

        plsc.subcore_barrier()

        def g_body(g, carry):
            pltpu.async_copy(ones_v, counts_sh.at[idx_v.at[g]], sem, add=True)
            return carry

        lax.fori_loop(0, NCHUNK, g_body, 0)

        def d_body(g, carry):
            pltpu.make_async_copy(
                ones_v, counts_sh.at[idx_v.at[0]], sem).wait()
            return carry

        lax.fori_loop(0, NCHUNK, d_body, 0)
        plsc.subcore_barrier()

        @pl.when(sid == 0)
        def _():
            pltpu.sync_copy(counts_sh, out_hbm.at[cid])

    return body(x2d, zeros_hbm)


LNB = 32768


def _tc_dot(h2, vt, w, b2d):
    nbins = vt.shape[1]
    nblk = pl.cdiv(nbins, LNB)

    nsl = LNB // 128

    def body(h_ref, vt_ref, w_ref, b_ref, o_ref, acc_ref):
        i = pl.program_id(0)

        @pl.when(i == 0)
        def _():
            acc_ref[...] = jnp.zeros_like(acc_ref)

        def accum(masked):
            ps = [jnp.zeros((DIM, 128), jnp.float32) for _ in range(2)]
            for k in range(nsl):
                sl = slice(k * 128, (k + 1) * 128)
                term = vt_ref[:, sl] * (h_ref[0:1, sl] + h_ref[1:2, sl])
                if masked:
                    lane = (lax.broadcasted_iota(jnp.int32, (DIM, 128), 1)
                            + (i * LNB + k * 128))
                    term = jnp.where(lane < nbins, term, 0.0)
                ps[k % 2] = ps[k % 2] + term
            acc_ref[...] += ps[0] + ps[1]

        @pl.when(i != nblk - 1)
        def _():
            accum(False)

        @pl.when(i == nblk - 1)
        def _():
            accum(True)
            vsum = jnp.sum(acc_ref[...], axis=1, keepdims=True)
            hrelu = jnp.maximum(vsum, 0.0)
            logits = lax.dot_general(
                w_ref[...], hrelu, (((1,), (0,)), ((), ())),
                precision=lax.Precision.HIGHEST,
                preferred_element_type=jnp.float32,
            ) + b_ref[...]
            m = jnp.max(logits, axis=0, keepdims=True)
            lse = m + jnp.log(
                jnp.sum(jnp.exp(logits - m), axis=0, keepdims=True))
            o_ref[...] = logits - lse

    return pl.pallas_call(
        body,
        grid=(nblk,),
        in_specs=[
            pl.BlockSpec((2, LNB), lambda i: (0, i)),
            pl.BlockSpec((DIM, LNB), lambda i: (0, i)),
            pl.BlockSpec((2, DIM), lambda i: (0, 0)),
            pl.BlockSpec((2, 1), lambda i: (0, 0)),
        ],
        out_specs=pl.BlockSpec((2, 1), lambda i: (0, 0)),
        out_shape=jax.ShapeDtypeStruct((2, 1), jnp.float32),
        scratch_shapes=[pltpu.VMEM((DIM, 128), jnp.float32)],
        compiler_params=pltpu.CompilerParams(
            vmem_limit_bytes=56 * 1024 * 1024),
    )(h2, vt, w, b2d)


def kernel(X, V, W, b):
    nbins = V.shape[0]
    x2d = X.reshape(NW * NCHUNK, CHUNK)
    zeros = jnp.zeros((nbins,), jnp.float32)
    h2 = _sc_hist(x2d, zeros, nbins)
    out = _tc_dot(h2, V.T, W, b.reshape(2, 1))
    return out.reshape(2)


# speedup vs baseline: 1.0209x; 1.0045x over previous
import functools

import jax
import jax.numpy as jnp
from jax import lax
from jax.experimental import pallas as pl
from jax.experimental.pallas import tpu as pltpu
from jax.experimental.pallas import tpu_sc as plsc

DIM = 64
N_TOK = 819200
NW = 32
BW = N_TOK // NW
CHUNK = 128
NCHUNK = BW // CHUNK


def _sc_hist(x2d, zeros_hbm, nbins):
    mesh = plsc.VectorSubcoreMesh(core_axis_name="c", subcore_axis_name="s")

    @functools.partial(
        pl.kernel,
        mesh=mesh,
        out_type=jax.ShapeDtypeStruct((2, nbins), jnp.float32),
        scratch_types=[
            pltpu.VMEM((NCHUNK, CHUNK), jnp.int32),
            pltpu.VMEM((CHUNK,), jnp.float32),
            pltpu.VMEM_SHARED((nbins,), jnp.float32),
            pltpu.SemaphoreType.DMA,
        ],
        compiler_params=pltpu.CompilerParams(use_tc_tiling_on_sc=False),
    )
    def body(x_hbm, z_hbm, out_hbm, idx_v, ones_v, counts_sh, sem):
        cid = lax.axis_index("c")
        sid = lax.axis_index("s")
        wid = sid * 2 + cid
        pltpu.sync_copy(x_hbm.at[pl.ds(wid * NCHUNK, NCHUNK)], idx_v)
        for c in range(CHUNK // 16):
            ones_v[pl.ds(c * 16, 16)] = jnp.ones((16,), jnp.float32)

        @pl.when(sid == 0)
        def _():
            pltpu.sync_copy(z_hbm, counts_sh)

        plsc.subcore_barrier()

        def g_body(g, carry):
            pltpu.async_copy(ones_v, counts_sh.at[idx_v.at[g]], sem, add=True)
            return carry

        lax.fori_loop(0, NCHUNK, g_body, 0)

        def d_body(g, carry):
            pltpu.make_async_copy(
                ones_v, counts_sh.at[idx_v.at[0]], sem).wait()
            return carry

        lax.fori_loop(0, NCHUNK, d_body, 0)
        plsc.subcore_barrier()

        @pl.when(sid == 0)
        def _():
            pltpu.sync_copy(counts_sh, out_hbm.at[cid])

    return body(x2d, zeros_hbm)


LNB = 32768


def _tc_dot(h2, vt, w, b2d):
    nbins = vt.shape[1]
    nblk = pl.cdiv(nbins, LNB)

    nsl = LNB // 128

    def body(h_ref, vt_ref, w_ref, b_ref, o_ref, acc_ref):
        i = pl.program_id(0)

        @pl.when(i == 0)
        def _():
            acc_ref[...] = jnp.zeros_like(acc_ref)

        def accum(masked):
            ps = [jnp.zeros((DIM, 128), jnp.float32) for _ in range(2)]
            for k in range(nsl):
                sl = slice(k * 128, (k + 1) * 128)
                term = vt_ref[:, sl] * (h_ref[0:1, sl] + h_ref[1:2, sl])
                if masked:
                    lane = (lax.broadcasted_iota(jnp.int32, (DIM, 128), 1)
                            + (i * LNB + k * 128))
                    term = jnp.where(lane < nbins, term, 0.0)
                ps[k % 2] = ps[k % 2] + term
            acc_ref[...] += ps[0] + ps[1]

        @pl.when(i != nblk - 1)
        def _():
            accum(False)

        @pl.when(i == nblk - 1)
        def _():
            accum(True)
            vsum = jnp.sum(acc_ref[...], axis=1, keepdims=True)
            hrelu = jnp.maximum(vsum, 0.0)
            logits = lax.dot_general(
                w_ref[...], hrelu, (((1,), (0,)), ((), ())),
                precision=lax.Precision.HIGHEST,
                preferred_element_type=jnp.float32,
            ) + b_ref[...]
            m = jnp.max(logits, axis=0, keepdims=True)
            lse = m + jnp.log(
                jnp.sum(jnp.exp(logits - m), axis=0, keepdims=True))
            o_ref[...] = logits - lse

    return pl.pallas_call(
        body,
        grid=(nblk,),
        in_specs=[
            pl.BlockSpec((2, LNB), lambda i: (0, i)),
            pl.BlockSpec((DIM, LNB), lambda i: (0, i)),
            pl.BlockSpec((2, DIM), lambda i: (0, 0)),
            pl.BlockSpec((2, 1), lambda i: (0, 0)),
        ],
        out_specs=pl.BlockSpec((2, 1), lambda i: (0, 0)),
        out_shape=jax.ShapeDtypeStruct((2, 1), jnp.float32),
        scratch_shapes=[pltpu.VMEM((DIM, 128), jnp.float32)],
        compiler_params=pltpu.CompilerParams(
            vmem_limit_bytes=56 * 1024 * 1024),
    )(h2, vt, w, b2d)


def kernel(X, V, W, b):
    nbins = V.shape[0]
    x2d = X.reshape(NW * NCHUNK, CHUNK)
    zeros = jnp.zeros((nbins,), jnp.float32)
    h2 = _sc_hist(x2d, zeros, nbins)
    out = _tc_dot(h2, V.T, W, b.reshape(2, 1))
    return out.reshape(2)
